# baseline (speedup 1.0000x reference)
import functools
import math

import jax
import jax.numpy as jnp
from jax.experimental import pallas as pl

H = 128
H3 = 384


def _xh_body(x_ref, g_ref, b_ref, w1t_ref, b1_ref, w2t_ref, b2_ref, o_ref):
    x = x_ref[...]
    mu = jnp.mean(x, axis=1, keepdims=True)
    var = jnp.mean((x - mu) ** 2, axis=1, keepdims=True)
    xln = (x - mu) * jax.lax.rsqrt(var + 1e-5) * g_ref[...] + b_ref[...]
    h = xln @ w1t_ref[...] + b1_ref[...]
    h = (h * jax.nn.sigmoid(h)) * (1.0 / 0.6)
    o_ref[...] = h @ w2t_ref[...] + b2_ref[...]


def _compute_xh(x, ln_g, ln_b, W1, b1, W2, b2):
    n = x.shape[0]
    blk = 256
    grid = (n + blk - 1) // blk
    return pl.pallas_call(
        _xh_body,
        grid=(grid,),
        in_specs=[
            pl.BlockSpec((blk, H), lambda i: (i, 0)),
            pl.BlockSpec((1, H), lambda i: (0, 0)),
            pl.BlockSpec((1, H), lambda i: (0, 0)),
            pl.BlockSpec((H, H), lambda i: (0, 0)),
            pl.BlockSpec((1, H), lambda i: (0, 0)),
            pl.BlockSpec((H, H3), lambda i: (0, 0)),
            pl.BlockSpec((1, H3), lambda i: (0, 0)),
        ],
        out_specs=pl.BlockSpec((blk, H3), lambda i: (i, 0)),
        out_shape=jax.ShapeDtypeStruct((grid * blk, H3), jnp.float32),
    )(
        x,
        ln_g.reshape(1, H),
        ln_b.reshape(1, H),
        W1.T,
        b1.reshape(1, H),
        W2.T,
        b2.reshape(1, H3),
    )[:n]


def _rbf_body(rbf_ref, wrt_ref, brs_ref, ev_ref, o_ref):
    mm = rbf_ref[...] @ wrt_ref[...] + brs_ref[...]
    ev = ev_ref[...]
    pad = jnp.zeros((mm.shape[0], 13), jnp.float32)
    o_ref[...] = jnp.concatenate([mm, ev, pad], axis=1)


def _compute_rbfE(edge_rbf, edge_vector, Wr, br):
    e = edge_rbf.shape[0]
    inv3 = 1.0 / math.sqrt(3.0)
    invh = 1.0 / math.sqrt(H)
    s = jnp.concatenate(
        [jnp.ones((H,)), jnp.full((H,), inv3 * invh), jnp.full((H,), invh)]
    ).astype(jnp.float32)
    wrt = Wr.T * s[None, :]
    brs = (br * s).reshape(1, H3)
    blk = 512
    grid = e // blk
    return pl.pallas_call(
        _rbf_body,
        grid=(grid,),
        in_specs=[
            pl.BlockSpec((blk, H), lambda i: (i, 0)),
            pl.BlockSpec((H, H3), lambda i: (0, 0)),
            pl.BlockSpec((1, H3), lambda i: (0, 0)),
            pl.BlockSpec((blk, 3), lambda i: (i, 0)),
        ],
        out_specs=pl.BlockSpec((blk, 400), lambda i: (i, 0)),
        out_shape=jax.ShapeDtypeStruct((e, 400), jnp.float32),
    )(edge_rbf, wrt, brs, edge_vector)


def kernel(x, vec, edge_index, edge_rbf, edge_vector, ln_g, ln_b, W1, b1, W2, b2, Wr, br):
    n = x.shape[0]
    xh = _compute_xh(x, ln_g, ln_b, W1, b1, W2, b2)
    rbfE = _compute_rbfE(edge_rbf, edge_vector, Wr, br)
    src = edge_index[0]
    dst = edge_index[1]
    m = jnp.take(xh, src, axis=0) * rbfE[:, :H3]
    xm = m[:, :H]
    m2 = m[:, H : 2 * H]
    m3 = m[:, 2 * H :]
    vec_j = jnp.take(vec.reshape(n, H3), src, axis=0)
    ev = rbfE[:, H3 : H3 + 3]
    vecm = vec_j.reshape(-1, 3, H) * m2[:, None, :] + m3[:, None, :] * ev[:, :, None]
    cnt = jax.ops.segment_sum(jnp.ones((m.shape[0],), jnp.float32), dst, num_segments=n)
    cnt = jnp.maximum(cnt, 1.0)
    dx = jax.ops.segment_sum(xm, dst, num_segments=n) / cnt[:, None]
    dvec = jax.ops.segment_sum(vecm, dst, num_segments=n) / cnt[:, None, None]
    return (dx, dvec)


# speedup vs baseline: 5.9064x; 5.9064x over previous
import functools
import math

import jax
import jax.numpy as jnp
from jax import lax
from jax.experimental import pallas as pl
from jax.experimental.pallas import tpu as pltpu
from jax.experimental.pallas import tpu_sc as plsc

H = 128
H3 = 384
N_NODES = 10000
N_EDGES = 320000
RW2 = 264
PW = 128
CNR = 80
NPAD = 10240
NSUB = 16
TILE_E = N_EDGES // 2 // NSUB
SEG = 400
NSEG = TILE_E // SEG
NB = SEG // 16
STRIPE = NPAD // NSUB
ZR = 8


def _xh_body(x_ref, g_ref, b_ref, w1t_ref, b1_ref, w2t_ref, b2_ref, o_ref):
    x = x_ref[...]
    mu = jnp.mean(x, axis=1, keepdims=True)
    var = jnp.mean((x - mu) ** 2, axis=1, keepdims=True)
    xln = (x - mu) * lax.rsqrt(var + 1e-5) * g_ref[...] + b_ref[...]
    h = xln @ w1t_ref[...] + b1_ref[...]
    h = (h * jax.nn.sigmoid(h)) * (1.0 / 0.6)
    o_ref[...] = h @ w2t_ref[...] + b2_ref[...]


def _compute_xh(x, ln_g, ln_b, W1, b1, W2, b2):
    n = x.shape[0]
    blk = 256
    grid = (n + blk - 1) // blk
    return pl.pallas_call(
        _xh_body,
        grid=(grid,),
        in_specs=[
            pl.BlockSpec((blk, H), lambda i: (i, 0)),
            pl.BlockSpec((1, H), lambda i: (0, 0)),
            pl.BlockSpec((1, H), lambda i: (0, 0)),
            pl.BlockSpec((H, H), lambda i: (0, 0)),
            pl.BlockSpec((1, H), lambda i: (0, 0)),
            pl.BlockSpec((H, H3), lambda i: (0, 0)),
            pl.BlockSpec((1, H3), lambda i: (0, 0)),
        ],
        out_specs=pl.BlockSpec((blk, H3), lambda i: (i, 0)),
        out_shape=jax.ShapeDtypeStruct((grid * blk, H3), jnp.float32),
    )(
        x,
        ln_g.reshape(1, H),
        ln_b.reshape(1, H),
        W1.T,
        b1.reshape(1, H),
        W2.T,
        b2.reshape(1, H3),
    )


def _rbf_body(rbf_ref, wrt_ref, brs_ref, ev_ref, o1_ref, o2_ref):
    mm = rbf_ref[...] @ wrt_ref[...] + brs_ref[...]
    ev = ev_ref[...]
    pad = jnp.zeros((mm.shape[0], RW2 - 2 * H - 3), jnp.float32)
    o1_ref[...] = mm[:, :H]
    o2_ref[...] = jnp.concatenate([mm[:, H:], ev, pad], axis=1)


def _compute_rbf(edge_rbf, edge_vector, Wr, br):
    e = edge_rbf.shape[0]
    inv3 = 1.0 / math.sqrt(3.0)
    invh = 1.0 / math.sqrt(H)
    s = jnp.concatenate(
        [jnp.ones((H,)), jnp.full((H,), inv3 * invh), jnp.full((H,), invh)]
    ).astype(jnp.float32)
    wrt = Wr.T * s[None, :]
    brs = (br * s).reshape(1, H3)
    blk = 512
    grid = e // blk
    return pl.pallas_call(
        _rbf_body,
        grid=(grid,),
        in_specs=[
            pl.BlockSpec((blk, H), lambda i: (i, 0)),
            pl.BlockSpec((H, H3), lambda i: (0, 0)),
            pl.BlockSpec((1, H3), lambda i: (0, 0)),
            pl.BlockSpec((blk, 3), lambda i: (i, 0)),
        ],
        out_specs=[
            pl.BlockSpec((blk, H), lambda i: (i, 0)),
            pl.BlockSpec((blk, RW2), lambda i: (i, 0)),
        ],
        out_shape=[
            jax.ShapeDtypeStruct((e, H), jnp.float32),
            jax.ShapeDtypeStruct((e, RW2), jnp.float32),
        ],
    )(edge_rbf, wrt, brs, edge_vector)


def _dx_body(xh1_hbm, rbf1_hbm, src_hbm, dst_hbm,
             o0, o1, oc0, oc1,
             sidx, didx, g1, rb1, pay, pay2, zb, acc, acc2, sem):
    core = lax.axis_index("c")
    sub = lax.axis_index("s")
    iota = lax.iota(jnp.int32, 16)
    zf = jnp.zeros((16,), jnp.float32)
    for zr in range(ZR):
        for zc in range(PW // 16):
            zb[zr, pl.ds(zc * 16, 16)] = zf
    ebase = core * (N_EDGES // 2) + sub * TILE_E
    srow = sub * STRIPE

    def zstep(z, carry):
        pltpu.sync_copy(zb, acc.at[pl.ds(srow + z * ZR, ZR)])
        return carry
    lax.fori_loop(0, STRIPE // ZR, zstep, 0)

    @pl.when(sub == 0)
    def _():
        def zstep2(z, carry):
            pltpu.sync_copy(zb, acc2.at[pl.ds(z * ZR, ZR)])
            return carry
        lax.fori_loop(0, CNR // ZR, zstep2, 0)
    plsc.subcore_barrier()

    def seg_body(sg, carry):
        segbase = ebase + sg * SEG
        pltpu.sync_copy(src_hbm.at[pl.ds(segbase, SEG)], sidx)
        pltpu.sync_copy(dst_hbm.at[pl.ds(segbase, SEG)], didx)

        def batch(b, c_):
            b16 = b * 16
            sv = sidx[pl.ds(b16, 16)]
            dvv = didx[pl.ds(b16, 16)]
            c1 = pltpu.async_copy(xh1_hbm.at[sv], g1, sem)
            c2 = pltpu.async_copy(rbf1_hbm.at[pl.ds(segbase + b16, 16)],
                                  rb1, sem)
            c1.wait()
            c2.wait()
            for i in range(16):
                dsc = dvv[i]
                col = dsc & 127
                for hh in range(8):
                    o = hh * 16
                    pay[i, pl.ds(o, 16)] = (g1[i, pl.ds(o, 16)]
                                            * rb1[i, pl.ds(o, 16)])
                    pay2[i, pl.ds(o, 16)] = jnp.where(
                        iota + o == col, 1.0, 0.0)
            pltpu.sync_copy(pay, acc.at[dvv], add=True)
            pltpu.sync_copy(pay2, acc2.at[lax.shift_right_logical(dvv, 7)],
                            add=True)
            return c_

        lax.fori_loop(0, NB, batch, 0)
        return carry

    lax.fori_loop(0, NSEG, seg_body, 0)
    plsc.subcore_barrier()

    @pl.when(core == 0)
    def _():
        def fstep(z, carry):
            r0 = srow + z * ZR
            pltpu.sync_copy(acc.at[pl.ds(r0, ZR)], o0.at[pl.ds(r0, ZR)])
            return carry
        lax.fori_loop(0, STRIPE // ZR, fstep, 0)

        @pl.when(sub == 0)
        def _():
            def cstep(z, carry):
                pltpu.sync_copy(acc2.at[pl.ds(z * ZR, ZR)],
                                oc0.at[pl.ds(z * ZR, ZR)])
                return carry
            lax.fori_loop(0, CNR // ZR, cstep, 0)

    @pl.when(core == 1)
    def _():
        def fstep(z, carry):
            r0 = srow + z * ZR
            pltpu.sync_copy(acc.at[pl.ds(r0, ZR)], o1.at[pl.ds(r0, ZR)])
            return carry
        lax.fori_loop(0, STRIPE // ZR, fstep, 0)

        @pl.when(sub == 0)
        def _():
            def cstep(z, carry):
                pltpu.sync_copy(acc2.at[pl.ds(z * ZR, ZR)],
                                oc1.at[pl.ds(z * ZR, ZR)])
                return carry
            lax.fori_loop(0, CNR // ZR, cstep, 0)


def _dvec_body(evlane, xh23_hbm, v_hbm, rbf23_hbm, src_hbm, dst_hbm,
               o0, o1,
               sidx, didx, g1, g2, rb2, pay, zb, acc, sem):
    core = lax.axis_index("c")
    sub = lax.axis_index("s")
    zf = jnp.zeros((16,), jnp.float32)
    for zr in range(ZR):
        for zc in range(PW // 16):
            zb[zr, pl.ds(zc * 16, 16)] = zf
    ebase = core * (N_EDGES // 2) + sub * TILE_E
    srow = sub * STRIPE

    def zstep(z, carry):
        pltpu.sync_copy(zb, acc.at[pl.ds(srow + z * ZR, ZR)])
        return carry
    lax.fori_loop(0, STRIPE // ZR, zstep, 0)
    plsc.subcore_barrier()

    def seg_body(sg, carry):
        segbase = ebase + sg * SEG
        pltpu.sync_copy(src_hbm.at[pl.ds(segbase, SEG)], sidx)
        pltpu.sync_copy(dst_hbm.at[pl.ds(segbase, SEG)], didx)

        def batch(b, c_):
            b16 = b * 16
            sv = sidx[pl.ds(b16, 16)]
            dvv = didx[pl.ds(b16, 16)]
            c1 = pltpu.async_copy(xh23_hbm.at[sv], g2, sem)
            c2 = pltpu.async_copy(v_hbm.at[sv], g1, sem)
            c3 = pltpu.async_copy(rbf23_hbm.at[pl.ds(segbase + b16, 16)],
                                  rb2, sem)
            c1.wait()
            c2.wait()
            c3.wait()
            for i in range(16):
                evc = rb2[i, pl.ds(248, 16)][evlane]
                for hh in range(8):
                    o = hh * 16
                    m2 = g2[i, pl.ds(o, 16)] * rb2[i, pl.ds(o, 16)]
                    m3 = g2[i, pl.ds(H + o, 16)] * rb2[i, pl.ds(H + o, 16)]
                    pay[i, pl.ds(o, 16)] = (g1[i, pl.ds(o, 16)] * m2
                                            + m3 * evc)
            pltpu.sync_copy(pay, acc.at[dvv], add=True)
            return c_

        lax.fori_loop(0, NB, batch, 0)
        return carry

    lax.fori_loop(0, NSEG, seg_body, 0)
    plsc.subcore_barrier()

    @pl.when(core == 0)
    def _():
        def fstep(z, carry):
            r0 = srow + z * ZR
            pltpu.sync_copy(acc.at[pl.ds(r0, ZR)], o0.at[pl.ds(r0, ZR)])
            return carry
        lax.fori_loop(0, STRIPE // ZR, fstep, 0)

    @pl.when(core == 1)
    def _():
        def fstep(z, carry):
            r0 = srow + z * ZR
            pltpu.sync_copy(acc.at[pl.ds(r0, ZR)], o1.at[pl.ds(r0, ZR)])
            return carry
        lax.fori_loop(0, STRIPE // ZR, fstep, 0)


def _mesh():
    return plsc.VectorSubcoreMesh(core_axis_name="c", subcore_axis_name="s",
                                  num_subcores=NSUB)


def _run_dx(xh1, rbf1, src, dst):
    slab = jax.ShapeDtypeStruct((NPAD, PW), jnp.float32)
    cslab = jax.ShapeDtypeStruct((CNR, PW), jnp.float32)
    f = functools.partial(
        pl.kernel,
        mesh=_mesh(),
        out_type=[slab, slab, cslab, cslab],
        scratch_types=[
            pltpu.VMEM((SEG,), jnp.int32),
            pltpu.VMEM((SEG,), jnp.int32),
            pltpu.VMEM((16, H), jnp.float32),
            pltpu.VMEM((16, H), jnp.float32),
            pltpu.VMEM((16, PW), jnp.float32),
            pltpu.VMEM((16, PW), jnp.float32),
            pltpu.VMEM((ZR, PW), jnp.float32),
            pltpu.VMEM_SHARED((NPAD, PW), jnp.float32),
            pltpu.VMEM_SHARED((CNR, PW), jnp.float32),
            pltpu.SemaphoreType.DMA,
        ],
    )(_dx_body)
    return f(xh1, rbf1, src, dst)


def _run_dvec(evlane, xh23, v, rbf23, src, dst):
    slab = jax.ShapeDtypeStruct((NPAD, PW), jnp.float32)
    f = functools.partial(
        pl.kernel,
        mesh=_mesh(),
        out_type=[slab, slab],
        scratch_types=[
            pltpu.VMEM((SEG,), jnp.int32),
            pltpu.VMEM((SEG,), jnp.int32),
            pltpu.VMEM((16, H), jnp.float32),
            pltpu.VMEM((16, 2 * H), jnp.float32),
            pltpu.VMEM((16, RW2), jnp.float32),
            pltpu.VMEM((16, PW), jnp.float32),
            pltpu.VMEM((ZR, PW), jnp.float32),
            pltpu.VMEM_SHARED((NPAD, PW), jnp.float32),
            pltpu.SemaphoreType.DMA,
        ],
    )(functools.partial(_dvec_body, evlane))
    return f(xh23, v, rbf23, src, dst)


def _sc_scatter(xh1, xh23, v0, v1, v2, rbf1, rbf23, src, dst):
    odx0, odx1, oc0, oc1 = _run_dx(xh1, rbf1, src, dst)
    ov00, ov01 = _run_dvec(8, xh23, v0, rbf23, src, dst)
    ov10, ov11 = _run_dvec(9, xh23, v1, rbf23, src, dst)
    ov20, ov21 = _run_dvec(10, xh23, v2, rbf23, src, dst)
    return [odx0, odx1, ov00, ov01, ov10, ov11, ov20, ov21, oc0, oc1]


def _combine_body(s0, s1, s2, s3, s4, s5, s6, s7, c0, c1, dx_ref, dv_ref):
    cnt = jnp.maximum(c0[:, 0:1] + c1[:, 0:1], 1.0)
    dx_ref[...] = (s0[...] + s1[...]) / cnt
    dv_ref[...] = jnp.concatenate(
        [(s2[...] + s3[...]) / cnt,
         (s4[...] + s5[...]) / cnt,
         (s6[...] + s7[...]) / cnt], axis=1)


def _combine(slabs):
    blk = 512
    grid = NPAD // blk
    spec = pl.BlockSpec((blk, H), lambda i: (i, 0))
    cspec = pl.BlockSpec((blk, 1), lambda i: (i, 0))
    return pl.pallas_call(
        _combine_body,
        grid=(grid,),
        in_specs=[spec] * 8 + [cspec] * 2,
        out_specs=[
            pl.BlockSpec((blk, H), lambda i: (i, 0)),
            pl.BlockSpec((blk, H3), lambda i: (i, 0)),
        ],
        out_shape=[
            jax.ShapeDtypeStruct((NPAD, H), jnp.float32),
            jax.ShapeDtypeStruct((NPAD, H3), jnp.float32),
        ],
    )(*slabs)


def kernel(x, vec, edge_index, edge_rbf, edge_vector, ln_g, ln_b, W1, b1, W2, b2, Wr, br):
    n = x.shape[0]
    xh = _compute_xh(x, ln_g, ln_b, W1, b1, W2, b2)[:n]
    xh1 = xh[:, :H]
    xh23 = xh[:, H:]
    rbf1, rbf23 = _compute_rbf(edge_rbf, edge_vector, Wr, br)
    vp = vec.transpose(1, 0, 2)
    src = edge_index[0]
    dst = edge_index[1]
    out = _sc_scatter(xh1, xh23, vp[0], vp[1], vp[2], rbf1, rbf23, src, dst)
    slabs = (list(out[:8])
             + [out[8].reshape(NPAD, 1), out[9].reshape(NPAD, 1)])
    dx, dvf = _combine(slabs)
    return (dx[:n], dvf[:n].reshape(n, 3, H))
